# baseline (speedup 1.0000x reference)
import functools

import jax
import jax.numpy as jnp
import numpy as np
from jax import lax
from jax.experimental import pallas as pl
from jax.experimental.pallas import tpu as pltpu
from jax.experimental.pallas import tpu_sc as plsc

_NUM_FIELDS = 26
_FIELD_DIM = 100000
_B = 16384
_NC = 2
_NS = 16
_NW = _NC * _NS
_BPW = _B // _NW
_CHUNK = 128
_NJ = _BPW // _CHUNK
_GRP = _NUM_FIELDS * _CHUNK
_L = 16

_OFFSETS = np.arange(_NUM_FIELDS, dtype=np.int32) * _FIELD_DIM


def _tec_body(x_hbm, w_hbm, out_hbm, *scratch):
    idx_vs = scratch[0:_NJ]
    val_vs = scratch[_NJ : 2 * _NJ]
    acc_v = scratch[2 * _NJ]
    sems = scratch[2 * _NJ + 1 :]

    wid = lax.axis_index("s") * _NC + lax.axis_index("c")
    base = wid * _BPW

    for j in range(_NJ):
        pltpu.sync_copy(x_hbm.at[wid, j], idx_vs[j])
        pltpu.make_async_copy(
            w_hbm.at[idx_vs[j]], val_vs[j], sems[j]
        ).start()

    for j in range(_NJ):
        pltpu.make_async_copy(
            w_hbm.at[idx_vs[j]], val_vs[j], sems[j]
        ).wait()

        def _red(c, carry, j=j):
            acc = val_vs[j][pl.ds(c * _L, _L)]
            for f in range(1, _NUM_FIELDS):
                acc = acc + val_vs[j][pl.ds(f * _CHUNK + c * _L, _L)]
            acc_v[pl.ds(j * _CHUNK + c * _L, _L)] = acc
            return carry

        lax.fori_loop(0, _CHUNK // _L, _red, 0)

    pltpu.sync_copy(acc_v, out_hbm.at[pl.ds(base, _BPW)])


_lookup = functools.partial(
    pl.kernel,
    out_type=jax.ShapeDtypeStruct((_B,), jnp.float32),
    mesh=plsc.VectorSubcoreMesh(
        core_axis_name="c", subcore_axis_name="s", num_cores=_NC
    ),
    scratch_types=(
        [pltpu.VMEM((_GRP,), jnp.int32) for _ in range(_NJ)]
        + [pltpu.VMEM((_GRP,), jnp.float32) for _ in range(_NJ)]
        + [pltpu.VMEM((_BPW,), jnp.float32)]
        + [pltpu.SemaphoreType.DMA for _ in range(_NJ)]
    ),
)(_tec_body)


@jax.jit
def kernel(x, W, bias):
    xt = (
        (x + jnp.asarray(_OFFSETS)[None, :])
        .T.reshape(_NUM_FIELDS, _NW, _NJ, _CHUNK)
        .transpose(1, 2, 0, 3)
        .reshape(_NW, _NJ, _GRP)
    )
    out = _lookup(xt, W.reshape(-1))
    return out[:, None] + bias[None, :]


# speedup vs baseline: 1.0093x; 1.0093x over previous
import functools

import jax
import jax.numpy as jnp
import numpy as np
from jax import lax
from jax.experimental import pallas as pl
from jax.experimental.pallas import tpu as pltpu
from jax.experimental.pallas import tpu_sc as plsc

_NUM_FIELDS = 26
_FIELD_DIM = 100000
_B = 16384
_NC = 2
_NS = 16
_NW = _NC * _NS
_BPW = _B // _NW
_SLAB = _NUM_FIELDS * _BPW
_L = 16

_OFFSETS = np.arange(_NUM_FIELDS, dtype=np.int32) * _FIELD_DIM


def _tec_body(x_hbm, w_hbm, out_hbm, idx_v, val_v, acc_v, sem):
    wid = lax.axis_index("s") * _NC + lax.axis_index("c")
    base = wid * _BPW

    pltpu.sync_copy(x_hbm.at[wid], idx_v)

    cp = pltpu.make_async_copy(w_hbm.at[idx_v], val_v, sem)
    cp.start()
    cp.wait()

    def _red(c, carry):
        acc = val_v[pl.ds(c * _L, _L)]
        for f in range(1, _NUM_FIELDS):
            acc = acc + val_v[pl.ds(f * _BPW + c * _L, _L)]
        acc_v[pl.ds(c * _L, _L)] = acc
        return carry

    lax.fori_loop(0, _BPW // _L, _red, 0)

    pltpu.sync_copy(acc_v, out_hbm.at[pl.ds(base, _BPW)])


_lookup = functools.partial(
    pl.kernel,
    out_type=jax.ShapeDtypeStruct((_B,), jnp.float32),
    mesh=plsc.VectorSubcoreMesh(
        core_axis_name="c", subcore_axis_name="s", num_cores=_NC
    ),
    scratch_types=[
        pltpu.VMEM((_SLAB,), jnp.int32),
        pltpu.VMEM((_SLAB,), jnp.float32),
        pltpu.VMEM((_BPW,), jnp.float32),
        pltpu.SemaphoreType.DMA,
    ],
)(_tec_body)


@jax.jit
def kernel(x, W, bias):
    xt = (
        (x + jnp.asarray(_OFFSETS)[None, :])
        .T.reshape(_NUM_FIELDS, _NW, _BPW)
        .transpose(1, 0, 2)
        .reshape(_NW, _SLAB)
    )
    out = _lookup(xt, W.reshape(-1))
    return out[:, None] + bias[None, :]


# speedup vs baseline: 1.0111x; 1.0019x over previous
import functools

import jax
import jax.numpy as jnp
import numpy as np
from jax import lax
from jax.experimental import pallas as pl
from jax.experimental.pallas import tpu as pltpu
from jax.experimental.pallas import tpu_sc as plsc

_NUM_FIELDS = 26
_FIELD_DIM = 100000
_B = 16384
_NC = 2
_NS = 16
_NW = _NC * _NS
_BPW = _B // _NW
_SLAB = _NUM_FIELDS * _BPW
_L = 16

_OFFSETS = np.arange(_NUM_FIELDS, dtype=np.int32) * _FIELD_DIM


def _tec_body(x_hbm, w_hbm, out_hbm, idx_v, val_v, acc_v, sem, sem2):
    wid = lax.axis_index("s") * _NC + lax.axis_index("c")
    base = wid * _BPW

    pltpu.sync_copy(x_hbm.at[wid], idx_v)

    h = _SLAB // 2
    cp0 = pltpu.make_async_copy(
        w_hbm.at[idx_v.at[pl.ds(0, h)]], val_v.at[pl.ds(0, h)], sem
    )
    cp1 = pltpu.make_async_copy(
        w_hbm.at[idx_v.at[pl.ds(h, h)]], val_v.at[pl.ds(h, h)], sem2
    )
    cp0.start()
    cp1.start()
    cp0.wait()
    cp1.wait()

    def _red(c, carry):
        acc = val_v[pl.ds(c * _L, _L)]
        for f in range(1, _NUM_FIELDS):
            acc = acc + val_v[pl.ds(f * _BPW + c * _L, _L)]
        acc_v[pl.ds(c * _L, _L)] = acc
        return carry

    lax.fori_loop(0, _BPW // _L, _red, 0)

    pltpu.sync_copy(acc_v, out_hbm.at[pl.ds(base, _BPW)])


_lookup = functools.partial(
    pl.kernel,
    out_type=jax.ShapeDtypeStruct((_B,), jnp.float32),
    mesh=plsc.VectorSubcoreMesh(
        core_axis_name="c", subcore_axis_name="s", num_cores=_NC
    ),
    scratch_types=[
        pltpu.VMEM((_SLAB,), jnp.int32),
        pltpu.VMEM((_SLAB,), jnp.float32),
        pltpu.VMEM((_BPW,), jnp.float32),
        pltpu.SemaphoreType.DMA,
        pltpu.SemaphoreType.DMA,
    ],
)(_tec_body)


@jax.jit
def kernel(x, W, bias):
    xt = (
        (x + jnp.asarray(_OFFSETS)[None, :])
        .T.reshape(_NUM_FIELDS, _NW, _BPW)
        .transpose(1, 0, 2)
        .reshape(_NW, _SLAB)
    )
    out = _lookup(xt, W.reshape(-1))
    return out[:, None] + bias[None, :]
